# baseline (speedup 1.0000x reference)
import functools

import jax
import jax.numpy as jnp
from jax import lax
from jax.experimental import pallas as pl
from jax.experimental.pallas import tpu as pltpu
from jax.experimental.pallas import tpu_sc as plsc

E = 65536
N = 2048
EDGE_DIM = 16
H = 8
KEYS = N * N
OUTSZ = N * N * H
NC = 2
NS = 16
NW = NC * NS
RANGE = KEYS // NW
SLAB = OUTSZ // NW
ZSZ = 8192
KCH = 8192
MCAP = 8192
BM = 2048
NBMAX = MCAP // BM
SUBR = 32768


def _mlp_body(ei_ref, x_ref, w1_ref, b1_ref, w2_ref, b2_ref,
              ebt_ref, key_ref):
    h = jnp.maximum(
        jnp.dot(x_ref[...], w1_ref[...], preferred_element_type=jnp.float32)
        + b1_ref[...], 0.0)
    ebt_ref[...] = (
        jnp.dot(w2_ref[...].T, h.T, preferred_element_type=jnp.float32)
        + b2_ref[...])
    k = ei_ref[0, :] * N + ei_ref[1, :]
    key_ref[...] = k.reshape(key_ref.shape)


_G = 8
_EB = E // _G


_mlp_call = pl.pallas_call(
    _mlp_body,
    grid=(_G,),
    in_specs=[
        pl.BlockSpec((2, _EB), lambda g: (0, g)),
        pl.BlockSpec((_EB, EDGE_DIM), lambda g: (g, 0)),
        pl.BlockSpec((EDGE_DIM, EDGE_DIM), lambda g: (0, 0)),
        pl.BlockSpec((1, EDGE_DIM), lambda g: (0, 0)),
        pl.BlockSpec((EDGE_DIM, H), lambda g: (0, 0)),
        pl.BlockSpec((H, 1), lambda g: (0, 0)),
    ],
    out_specs=[
        pl.BlockSpec((H, _EB), lambda g: (0, g)),
        pl.BlockSpec((_EB // 128, 128), lambda g: (g, 0)),
    ],
    out_shape=[
        jax.ShapeDtypeStruct((H, E), jnp.float32),
        jax.ShapeDtypeStruct((E // 128, 128), jnp.int32),
    ],
)


_SH16 = 11


def _midx(pos):
    sh = jnp.full((16,), _SH16, jnp.int32)
    mskc = jnp.full((16,), BM - 1, jnp.int32)
    return [pos >> sh, pos & mskc]


def _sc_body(keys_hbm, ebt_hbm, zeros_hbm, out_hbm,
             zbuf, kbuf, mkeys, mids, gidx, sidx, vals, tagv,
             zsem, gsem, ssem):
    cid = lax.axis_index("c")
    sid = lax.axis_index("s")
    wid = sid * NC + cid
    lo = wid * RANGE
    hi = lo + RANGE
    slab0 = wid * SLAB

    pltpu.sync_copy(zeros_hbm, zbuf)

    def fire_zero(i, _):
        pltpu.make_async_copy(
            zbuf, out_hbm.at[pl.ds(slab0 + i * ZSZ, ZSZ)], zsem).start()
        return 0

    lax.fori_loop(0, SLAB // ZSZ, fire_zero, 0)

    def drain_zero(i, _):
        pltpu.make_async_copy(
            zbuf, out_hbm.at[pl.ds(slab0 + i * ZSZ, ZSZ)], zsem).wait()
        return 0

    iota = lax.broadcasted_iota(jnp.int32, (16,), 0)

    lo16 = jnp.full((16,), lo, jnp.int32)
    hi16 = jnp.full((16,), hi, jnp.int32)
    one16 = jnp.full((16,), 1, jnp.int32)
    cap16 = jnp.full((16,), MCAP - 1, jnp.int32)

    def scan_chunk(c, m):
        pltpu.sync_copy(keys_hbm.at[pl.ds(c * KCH, KCH)], kbuf)
        cbase = c * KCH

        def scan_vreg(j, m):
            k16 = kbuf[pl.ds(j * 16, 16)]
            msk = (k16 >= lo16) & (k16 < hi16)
            c16 = jnp.cumsum(msk.astype(jnp.int32))
            m16 = jnp.full((16,), m, jnp.int32)
            pos = jnp.minimum(m16 + c16 - one16, cap16)
            e16 = jnp.full((16,), cbase + j * 16, jnp.int32) + iota
            plsc.store_scatter(mkeys, _midx(pos), k16, mask=msk)
            plsc.store_scatter(mids, _midx(pos), e16, mask=msk)
            return jnp.minimum(m + jnp.max(c16), MCAP)

        return lax.fori_loop(0, KCH // 16, scan_vreg, m)

    m = lax.fori_loop(0, E // KCH, scan_chunk, jnp.int32(0))

    mm116 = jnp.full((16,), jnp.maximum(m - 1, 0), jnp.int32)
    zero16 = jnp.full((16,), 0, jnp.int32)
    sr16 = jnp.full((16,), SUBR, jnp.int32)
    nv = (m + 15) >> 4

    for s in range(RANGE // SUBR):
        sublo16 = jnp.full((16,), lo + s * SUBR, jnp.int32)
        subhi16 = sublo16 + sr16

        def tag_store(v, _):
            pos16 = jnp.full((16,), v * 16, jnp.int32) + iota
            valid = pos16 <= mm116
            pos_c = jnp.minimum(pos16, mm116)
            kk = plsc.load_gather(mkeys, _midx(pos_c))
            insub = valid & (kk >= sublo16) & (kk < subhi16)
            kidx = jnp.minimum(jnp.maximum(kk - sublo16, zero16),
                               sr16 - one16)
            plsc.store_scatter(tagv, [kidx], pos16, mask=insub)
            return 0

        def id_rewrite(v, _):
            pos16 = jnp.full((16,), v * 16, jnp.int32) + iota
            valid = pos16 <= mm116
            pos_c = jnp.minimum(pos16, mm116)
            kk = plsc.load_gather(mkeys, _midx(pos_c))
            insub = valid & (kk >= sublo16) & (kk < subhi16)
            kidx = jnp.minimum(jnp.maximum(kk - sublo16, zero16),
                               sr16 - one16)
            w16 = plsc.load_gather(tagv, [kidx])
            w_c = jnp.minimum(jnp.maximum(w16, zero16), mm116)
            wid16 = plsc.load_gather(mids, _midx(w_c))
            plsc.store_scatter(mids, _midx(pos_c), wid16, mask=insub)
            return 0

        lax.fori_loop(0, nv, tag_store, 0)
        lax.fori_loop(0, nv, id_rewrite, 0)

    key_last = plsc.load_gather(mkeys, _midx(mm116))
    id_last = plsc.load_gather(mids, _midx(mm116))
    nb = (m + BM - 1) >> 11

    def pad_slot(j, _):
        posv = j * 16 + iota
        pmsk = posv >= m
        plsc.store_scatter(mkeys, _midx(posv), key_last, mask=pmsk)
        plsc.store_scatter(mids, _midx(posv), id_last, mask=pmsk)
        return 0

    lax.fori_loop(m >> 4, nb << 7, pad_slot, 0)

    lax.fori_loop(0, SLAB // ZSZ, drain_zero, 0)

    c16k = jnp.full((16,), 16384, jnp.int32)
    c1k = jnp.full((16,), 1024, jnp.int32)
    sh11 = jnp.full((16,), 11, jnp.int32)
    sh7 = jnp.full((16,), 7, jnp.int32)
    m15 = jnp.full((16,), 15, jnp.int32)
    m127 = jnp.full((16,), 127, jnp.int32)

    def do_batch(b, _):
        def build_idx(v, _):
            sl = pl.ds(v * 16, 16)
            id16 = mids.at[b][sl]
            k16 = mkeys.at[b][sl]
            s0 = ((k16 >> sh11) * c16k + ((k16 >> sh7) & m15) * c1k
                  + (k16 & m127))
            for h in range(H):
                gidx.at[h][sl] = id16 + jnp.full((16,), h * E, jnp.int32)
                sidx.at[h][sl] = s0 + jnp.full((16,), h * 128, jnp.int32)
            return 0

        lax.fori_loop(0, BM // 16, build_idx, 0)

        for h in range(H):
            pltpu.make_async_copy(ebt_hbm.at[gidx.at[h]], vals.at[h],
                                  gsem).start()
        for h in range(H):
            pltpu.make_async_copy(ebt_hbm.at[gidx.at[h]], vals.at[h],
                                  gsem).wait()
        for h in range(H):
            pltpu.make_async_copy(vals.at[h], out_hbm.at[sidx.at[h]],
                                  ssem).start()
        for h in range(H):
            pltpu.make_async_copy(vals.at[h], out_hbm.at[sidx.at[h]],
                                  ssem).wait()
        return 0

    lax.fori_loop(0, nb, do_batch, 0)


_sc_call = functools.partial(
    pl.kernel,
    out_type=jax.ShapeDtypeStruct((OUTSZ,), jnp.float32),
    mesh=plsc.VectorSubcoreMesh(core_axis_name="c", subcore_axis_name="s"),
    compiler_params=pltpu.CompilerParams(
        needs_layout_passes=False, use_tc_tiling_on_sc=False),
    scratch_types=[
        pltpu.VMEM((ZSZ,), jnp.float32),
        pltpu.VMEM((KCH,), jnp.int32),
        pltpu.VMEM((NBMAX, BM), jnp.int32),
        pltpu.VMEM((NBMAX, BM), jnp.int32),
        pltpu.VMEM((H, BM), jnp.int32),
        pltpu.VMEM((H, BM), jnp.int32),
        pltpu.VMEM((H, BM), jnp.float32),
        pltpu.VMEM((SUBR,), jnp.int32),
        pltpu.SemaphoreType.DMA,
        pltpu.SemaphoreType.DMA,
        pltpu.SemaphoreType.DMA,
    ],
)(_sc_body)


def kernel(edge_index, edge_attr, num_nodes, W1, b1, W2, b2):
    ebt, keys2d = _mlp_call(
        edge_index, edge_attr, W1, b1.reshape(1, EDGE_DIM),
        W2, b2.reshape(H, 1))
    keys = keys2d.reshape(E)
    ebt_flat = ebt.reshape(H * E)
    zeros_in = jnp.zeros((ZSZ,), jnp.float32)
    out_flat = _sc_call(keys, ebt_flat, zeros_in)
    out4 = out_flat.reshape(N, N // 128, H, 128)
    return out4.transpose(0, 1, 3, 2).reshape(N, N, H)


# speedup vs baseline: 5.1581x; 5.1581x over previous
import functools

import jax
import jax.numpy as jnp
from jax import lax
from jax.experimental import pallas as pl
from jax.experimental.pallas import tpu as pltpu
from jax.experimental.pallas import tpu_sc as plsc

E = 65536
N = 2048
EDGE_DIM = 16
H = 8
KEYS = N * N
NC = 2
NS = 16
NW = NC * NS
RANGE = KEYS // NW
ZROWS = 512
KCH = 8192
MCAP = 8192
BM = 2048
NBMAX = MCAP // BM
SUBR = 16384
WELEMS = 16384
WR8 = WELEMS // 8
WR128 = WELEMS // 128
NWIN = RANGE * H // WELEMS


def _mlp_body(ei_ref, x_ref, w1_ref, b1_ref, w2_ref, b2_ref, eb_ref, key_ref):
    h = jnp.maximum(
        jnp.dot(x_ref[...], w1_ref[...], preferred_element_type=jnp.float32)
        + b1_ref[...], 0.0)
    eb_ref[...] = (
        jnp.dot(h, w2_ref[...], preferred_element_type=jnp.float32)
        + b2_ref[...])
    k = ei_ref[0, :] * N + ei_ref[1, :]
    key_ref[...] = k.reshape(key_ref.shape)


_G = 8
_EB = E // _G


_mlp_call = pl.pallas_call(
    _mlp_body,
    grid=(_G,),
    in_specs=[
        pl.BlockSpec((2, _EB), lambda g: (0, g)),
        pl.BlockSpec((_EB, EDGE_DIM), lambda g: (g, 0)),
        pl.BlockSpec((EDGE_DIM, EDGE_DIM), lambda g: (0, 0)),
        pl.BlockSpec((1, EDGE_DIM), lambda g: (0, 0)),
        pl.BlockSpec((EDGE_DIM, H), lambda g: (0, 0)),
        pl.BlockSpec((1, H), lambda g: (0, 0)),
    ],
    out_specs=[
        pl.BlockSpec((_EB, H), lambda g: (g, 0)),
        pl.BlockSpec((_EB // 128, 128), lambda g: (g, 0)),
    ],
    out_shape=[
        jax.ShapeDtypeStruct((E, H), jnp.float32),
        jax.ShapeDtypeStruct((E // 128, 128), jnp.int32),
    ],
)


_SH16 = 11


def _midx(pos):
    sh = jnp.full((16,), _SH16, jnp.int32)
    mskc = jnp.full((16,), BM - 1, jnp.int32)
    return [pos >> sh, pos & mskc]


def _sc_body(keys_hbm, ebias_hbm, zeros_hbm, scr_hbm, out_hbm,
             zbuf, kbuf, mkeys, mids, rows, tagv, wbuf, wtbuf,
             zsem, gsem, ssem, rsem, wsem):
    cid = lax.axis_index("c")
    sid = lax.axis_index("s")
    wid = sid * NC + cid
    lo = wid * RANGE
    hi = lo + RANGE

    pltpu.sync_copy(zeros_hbm, zbuf)

    def fire_zero(i, _):
        pltpu.make_async_copy(
            zbuf, scr_hbm.at[pl.ds(lo + i * ZROWS, ZROWS)], zsem).start()
        return 0

    lax.fori_loop(0, RANGE // ZROWS, fire_zero, 0)

    def drain_zero(i, _):
        pltpu.make_async_copy(
            zbuf, scr_hbm.at[pl.ds(lo + i * ZROWS, ZROWS)], zsem).wait()
        return 0

    iota = lax.broadcasted_iota(jnp.int32, (16,), 0)

    lo16 = jnp.full((16,), lo, jnp.int32)
    hi16 = jnp.full((16,), hi, jnp.int32)
    one16 = jnp.full((16,), 1, jnp.int32)
    cap16 = jnp.full((16,), MCAP - 1, jnp.int32)

    def scan_chunk(c, m):
        pltpu.sync_copy(keys_hbm.at[pl.ds(c * KCH, KCH)], kbuf)
        cbase = c * KCH

        def scan_vreg(j, m):
            k16 = kbuf[pl.ds(j * 16, 16)]
            msk = (k16 >= lo16) & (k16 < hi16)
            c16 = jnp.cumsum(msk.astype(jnp.int32))
            m16 = jnp.full((16,), m, jnp.int32)
            pos = jnp.minimum(m16 + c16 - one16, cap16)
            e16 = jnp.full((16,), cbase + j * 16, jnp.int32) + iota
            plsc.store_scatter(mkeys, _midx(pos), k16, mask=msk)
            plsc.store_scatter(mids, _midx(pos), e16, mask=msk)
            return jnp.minimum(m + jnp.max(c16), MCAP)

        return lax.fori_loop(0, KCH // 16, scan_vreg, m)

    m = lax.fori_loop(0, E // KCH, scan_chunk, jnp.int32(0))

    mm116 = jnp.full((16,), jnp.maximum(m - 1, 0), jnp.int32)
    zero16 = jnp.full((16,), 0, jnp.int32)
    sr16 = jnp.full((16,), SUBR, jnp.int32)
    nv = (m + 15) >> 4

    for s in range(RANGE // SUBR):
        sublo16 = jnp.full((16,), lo + s * SUBR, jnp.int32)
        subhi16 = sublo16 + sr16

        def tag_store(v, _):
            pos16 = jnp.full((16,), v * 16, jnp.int32) + iota
            valid = pos16 <= mm116
            pos_c = jnp.minimum(pos16, mm116)
            kk = plsc.load_gather(mkeys, _midx(pos_c))
            insub = valid & (kk >= sublo16) & (kk < subhi16)
            kidx = jnp.minimum(jnp.maximum(kk - sublo16, zero16),
                               sr16 - one16)
            plsc.store_scatter(tagv, [kidx], pos16, mask=insub)
            return 0

        def id_rewrite(v, _):
            pos16 = jnp.full((16,), v * 16, jnp.int32) + iota
            valid = pos16 <= mm116
            pos_c = jnp.minimum(pos16, mm116)
            kk = plsc.load_gather(mkeys, _midx(pos_c))
            insub = valid & (kk >= sublo16) & (kk < subhi16)
            kidx = jnp.minimum(jnp.maximum(kk - sublo16, zero16),
                               sr16 - one16)
            w16 = plsc.load_gather(tagv, [kidx])
            w_c = jnp.minimum(jnp.maximum(w16, zero16), mm116)
            wid16 = plsc.load_gather(mids, _midx(w_c))
            plsc.store_scatter(mids, _midx(pos_c), wid16, mask=insub)
            return 0

        lax.fori_loop(0, nv, tag_store, 0)
        lax.fori_loop(0, nv, id_rewrite, 0)

    key_last = plsc.load_gather(mkeys, _midx(mm116))
    id_last = plsc.load_gather(mids, _midx(mm116))
    nb = (m + BM - 1) >> 11

    def pad_slot(j, _):
        posv = j * 16 + iota
        pmsk = posv >= m
        plsc.store_scatter(mkeys, _midx(posv), key_last, mask=pmsk)
        plsc.store_scatter(mids, _midx(posv), id_last, mask=pmsk)
        return 0

    lax.fori_loop(m >> 4, nb << 7, pad_slot, 0)

    lax.fori_loop(0, RANGE // ZROWS, drain_zero, 0)

    def do_batch(b, _):
        g = pltpu.make_async_copy(ebias_hbm.at[mids.at[b]], rows, gsem)
        g.start()
        g.wait()
        s = pltpu.make_async_copy(rows, scr_hbm.at[mkeys.at[b]], ssem)
        s.start()
        s.wait()
        return 0

    lax.fori_loop(0, nb, do_batch, 0)

    r8_0 = lo
    r128_0 = lo // 16
    patt16 = (iota & jnp.full((16,), 7, jnp.int32)) * jnp.full(
        (16,), 128, jnp.int32) + (iota >> jnp.full((16,), 3, jnp.int32))
    iod8 = iota >> jnp.full((16,), 3, jnp.int32)
    ioc7 = iota & jnp.full((16,), 7, jnp.int32)
    sh7 = jnp.full((16,), 7, jnp.int32)
    m127 = jnp.full((16,), 127, jnp.int32)

    def read_win(w, buf):
        return pltpu.make_async_copy(
            scr_hbm.at[pl.ds(r8_0 + w * WR8, WR8)], buf, rsem)

    def write_win(w, buf):
        return pltpu.make_async_copy(
            buf, out_hbm.at[pl.ds(r128_0 + w * WR128, WR128)], wsem)

    read_win(0, wbuf.at[0]).start()

    def do_window(w, _):
        wb = wbuf.at[w & 1]
        wt = wtbuf.at[w & 1]
        read_win(w, wb).wait()

        @pl.when(w + 1 < NWIN)
        def _():
            read_win(w + 1, wbuf.at[(w + 1) & 1]).start()

        @pl.when(w >= 2)
        def _():
            write_win(w - 2, wtbuf.at[w & 1]).wait()

        def tx_vreg(v, _):
            r16 = jnp.full((16,), v * 2, jnp.int32) + iod8
            x16 = plsc.load_gather(wb, [r16, ioc7])
            base = (v >> 6) * 1024 + (v & 63) * 2
            idx16 = patt16 + jnp.full((16,), base, jnp.int32)
            plsc.store_scatter(wt, [idx16 >> sh7, idx16 & m127], x16)
            return 0

        lax.fori_loop(0, WELEMS // 16, tx_vreg, 0)
        write_win(w, wt).start()
        return 0

    lax.fori_loop(0, NWIN, do_window, 0)
    write_win(NWIN - 2, wtbuf.at[NWIN & 1]).wait()
    write_win(NWIN - 1, wtbuf.at[(NWIN - 1) & 1]).wait()


_sc_call = functools.partial(
    pl.kernel,
    out_type=[
        jax.ShapeDtypeStruct((KEYS, H), jnp.float32),
        jax.ShapeDtypeStruct((KEYS * H // 128, 128), jnp.float32),
    ],
    mesh=plsc.VectorSubcoreMesh(core_axis_name="c", subcore_axis_name="s"),
    compiler_params=pltpu.CompilerParams(
        needs_layout_passes=False, use_tc_tiling_on_sc=False),
    scratch_types=[
        pltpu.VMEM((ZROWS, H), jnp.float32),
        pltpu.VMEM((KCH,), jnp.int32),
        pltpu.VMEM((NBMAX, BM), jnp.int32),
        pltpu.VMEM((NBMAX, BM), jnp.int32),
        pltpu.VMEM((BM, H), jnp.float32),
        pltpu.VMEM((SUBR,), jnp.int32),
        pltpu.VMEM((2, WR8, H), jnp.float32),
        pltpu.VMEM((2, WR128, 128), jnp.float32),
        pltpu.SemaphoreType.DMA,
        pltpu.SemaphoreType.DMA,
        pltpu.SemaphoreType.DMA,
        pltpu.SemaphoreType.DMA,
        pltpu.SemaphoreType.DMA,
    ],
)(_sc_body)


def kernel(edge_index, edge_attr, num_nodes, W1, b1, W2, b2):
    ebias, keys2d = _mlp_call(
        edge_index, edge_attr, W1, b1.reshape(1, EDGE_DIM),
        W2, b2.reshape(1, H))
    keys = keys2d.reshape(E)
    zeros_in = jnp.zeros((ZROWS, H), jnp.float32)
    _, t_out = _sc_call(keys, ebias, zeros_in)
    out4 = t_out.reshape(N, N // 128, H, 128)
    return out4.transpose(0, 1, 3, 2).reshape(N, N, H)


# speedup vs baseline: 7.6761x; 1.4882x over previous
import functools

import jax
import jax.numpy as jnp
from jax import lax
from jax.experimental import pallas as pl
from jax.experimental.pallas import tpu as pltpu
from jax.experimental.pallas import tpu_sc as plsc

E = 65536
N = 2048
EDGE_DIM = 16
H = 8
KEYS = N * N
NC = 2
NS = 16
NW = NC * NS
RANGE = KEYS // NW
ZROWS = 512
KCH = 8192
MCAP = 8192
BM = 2048
NBMAX = MCAP // BM
SUBR = 16384
WELEMS = 16384
WR8 = WELEMS // 8
WR128 = WELEMS // 128
NWIN = RANGE * H // WELEMS


def _mlp_body(ei_ref, x_ref, w1_ref, b1_ref, w2_ref, b2_ref, eb_ref, key_ref):
    h = jnp.maximum(
        jnp.dot(x_ref[...], w1_ref[...], preferred_element_type=jnp.float32)
        + b1_ref[...], 0.0)
    eb_ref[...] = (
        jnp.dot(h, w2_ref[...], preferred_element_type=jnp.float32)
        + b2_ref[...])
    k = ei_ref[0, :] * N + ei_ref[1, :]
    key_ref[...] = k.reshape(key_ref.shape)


_G = 8
_EB = E // _G


_mlp_call = pl.pallas_call(
    _mlp_body,
    grid=(_G,),
    in_specs=[
        pl.BlockSpec((2, _EB), lambda g: (0, g)),
        pl.BlockSpec((_EB, EDGE_DIM), lambda g: (g, 0)),
        pl.BlockSpec((EDGE_DIM, EDGE_DIM), lambda g: (0, 0)),
        pl.BlockSpec((1, EDGE_DIM), lambda g: (0, 0)),
        pl.BlockSpec((EDGE_DIM, H), lambda g: (0, 0)),
        pl.BlockSpec((1, H), lambda g: (0, 0)),
    ],
    out_specs=[
        pl.BlockSpec((_EB, H), lambda g: (g, 0)),
        pl.BlockSpec((_EB // 128, 128), lambda g: (g, 0)),
    ],
    out_shape=[
        jax.ShapeDtypeStruct((E, H), jnp.float32),
        jax.ShapeDtypeStruct((E // 128, 128), jnp.int32),
    ],
)


_SH16 = 11


def _midx(pos):
    sh = jnp.full((16,), _SH16, jnp.int32)
    mskc = jnp.full((16,), BM - 1, jnp.int32)
    return [pos >> sh, pos & mskc]


def _sc_body(keys_hbm, ebias_hbm, zeros_hbm, scr_hbm, out_hbm,
             zbuf, kbuf, mkeys, mids, rows, tagv, wbuf, wtbuf,
             zsem, gsem, ssem, rsem, wsem):
    cid = lax.axis_index("c")
    sid = lax.axis_index("s")
    wid = sid * NC + cid
    lo = wid * RANGE
    hi = lo + RANGE

    pltpu.sync_copy(zeros_hbm, zbuf)

    def fire_zero(i, _):
        pltpu.make_async_copy(
            zbuf, scr_hbm.at[pl.ds(lo + i * ZROWS, ZROWS)], zsem).start()
        return 0

    lax.fori_loop(0, RANGE // ZROWS, fire_zero, 0)

    def drain_zero(i, _):
        pltpu.make_async_copy(
            zbuf, scr_hbm.at[pl.ds(lo + i * ZROWS, ZROWS)], zsem).wait()
        return 0

    iota = lax.broadcasted_iota(jnp.int32, (16,), 0)

    lo16 = jnp.full((16,), lo, jnp.int32)
    hi16 = jnp.full((16,), hi, jnp.int32)
    one16 = jnp.full((16,), 1, jnp.int32)
    cap16 = jnp.full((16,), MCAP - 1, jnp.int32)

    def scan_chunk(c, m):
        pltpu.sync_copy(keys_hbm.at[pl.ds(c * KCH, KCH)], kbuf)
        cbase = c * KCH

        def scan_vreg(j, m):
            k16 = kbuf[pl.ds(j * 16, 16)]
            msk = (k16 >= lo16) & (k16 < hi16)
            c16 = jnp.cumsum(msk.astype(jnp.int32))
            m16 = jnp.full((16,), m, jnp.int32)
            pos = jnp.minimum(m16 + c16 - one16, cap16)
            e16 = jnp.full((16,), cbase + j * 16, jnp.int32) + iota
            plsc.store_scatter(mkeys, _midx(pos), k16, mask=msk)
            plsc.store_scatter(mids, _midx(pos), e16, mask=msk)
            return jnp.minimum(m + jnp.max(c16), MCAP)

        return lax.fori_loop(0, KCH // 16, scan_vreg, m)

    m = lax.fori_loop(0, E // KCH, scan_chunk, jnp.int32(0))

    mm116 = jnp.full((16,), jnp.maximum(m - 1, 0), jnp.int32)
    zero16 = jnp.full((16,), 0, jnp.int32)
    sr16 = jnp.full((16,), SUBR, jnp.int32)
    nv = (m + 15) >> 4

    for s in range(RANGE // SUBR):
        sublo16 = jnp.full((16,), lo + s * SUBR, jnp.int32)
        subhi16 = sublo16 + sr16

        def tag_store(v, _):
            pos16 = jnp.full((16,), v * 16, jnp.int32) + iota
            valid = pos16 <= mm116
            pos_c = jnp.minimum(pos16, mm116)
            kk = plsc.load_gather(mkeys, _midx(pos_c))
            insub = valid & (kk >= sublo16) & (kk < subhi16)
            kidx = jnp.minimum(jnp.maximum(kk - sublo16, zero16),
                               sr16 - one16)
            plsc.store_scatter(tagv, [kidx], pos16, mask=insub)
            return 0

        def id_rewrite(v, _):
            pos16 = jnp.full((16,), v * 16, jnp.int32) + iota
            valid = pos16 <= mm116
            pos_c = jnp.minimum(pos16, mm116)
            kk = plsc.load_gather(mkeys, _midx(pos_c))
            insub = valid & (kk >= sublo16) & (kk < subhi16)
            kidx = jnp.minimum(jnp.maximum(kk - sublo16, zero16),
                               sr16 - one16)
            w16 = plsc.load_gather(tagv, [kidx])
            w_c = jnp.minimum(jnp.maximum(w16, zero16), mm116)
            wid16 = plsc.load_gather(mids, _midx(w_c))
            plsc.store_scatter(mids, _midx(pos_c), wid16, mask=insub)
            return 0

        lax.fori_loop(0, nv, tag_store, 0)
        lax.fori_loop(0, nv, id_rewrite, 0)

    key_last = plsc.load_gather(mkeys, _midx(mm116))
    id_last = plsc.load_gather(mids, _midx(mm116))
    nb = (m + BM - 1) >> 11

    def pad_slot(j, _):
        posv = j * 16 + iota
        pmsk = posv >= m
        plsc.store_scatter(mkeys, _midx(posv), key_last, mask=pmsk)
        plsc.store_scatter(mids, _midx(posv), id_last, mask=pmsk)
        return 0

    lax.fori_loop(m >> 4, nb << 7, pad_slot, 0)

    lax.fori_loop(0, RANGE // ZROWS, drain_zero, 0)

    def do_batch(b, _):
        g = pltpu.make_async_copy(ebias_hbm.at[mids.at[b]], rows, gsem)
        g.start()
        g.wait()
        s = pltpu.make_async_copy(rows, scr_hbm.at[mkeys.at[b]], ssem)
        s.start()
        s.wait()
        return 0

    lax.fori_loop(0, nb, do_batch, 0)

    r8_0 = lo
    r128_0 = lo // 16
    patt16 = (iota & jnp.full((16,), 7, jnp.int32)) * jnp.full(
        (16,), 128, jnp.int32) + (iota >> jnp.full((16,), 3, jnp.int32))
    iod8 = iota >> jnp.full((16,), 3, jnp.int32)
    ioc7 = iota & jnp.full((16,), 7, jnp.int32)
    sh7 = jnp.full((16,), 7, jnp.int32)
    m127 = jnp.full((16,), 127, jnp.int32)

    def read_win(w, buf):
        return pltpu.make_async_copy(
            scr_hbm.at[pl.ds(r8_0 + w * WR8, WR8)], buf, rsem)

    def write_win(w, buf):
        return pltpu.make_async_copy(
            buf, out_hbm.at[pl.ds(r128_0 + w * WR128, WR128)], wsem)

    read_win(0, wbuf.at[0]).start()

    def do_window(w, _):
        wb = wbuf.at[w & 1]
        wt = wtbuf.at[w & 1]
        read_win(w, wb).wait()

        @pl.when(w + 1 < NWIN)
        def _():
            read_win(w + 1, wbuf.at[(w + 1) & 1]).start()

        @pl.when(w >= 2)
        def _():
            write_win(w - 2, wtbuf.at[w & 1]).wait()

        @plsc.parallel_loop(0, WELEMS // 16, unroll=8)
        def tx_vreg(v):
            r16 = jnp.full((16,), v * 2, jnp.int32) + iod8
            x16 = plsc.load_gather(wb, [r16, ioc7])
            base = (v >> 6) * 1024 + (v & 63) * 2
            idx16 = patt16 + jnp.full((16,), base, jnp.int32)
            plsc.store_scatter(wt, [idx16 >> sh7, idx16 & m127], x16)

        write_win(w, wt).start()
        return 0

    lax.fori_loop(0, NWIN, do_window, 0)
    write_win(NWIN - 2, wtbuf.at[NWIN & 1]).wait()
    write_win(NWIN - 1, wtbuf.at[(NWIN - 1) & 1]).wait()


_sc_call = functools.partial(
    pl.kernel,
    out_type=[
        jax.ShapeDtypeStruct((KEYS, H), jnp.float32),
        jax.ShapeDtypeStruct((KEYS * H // 128, 128), jnp.float32),
    ],
    mesh=plsc.VectorSubcoreMesh(core_axis_name="c", subcore_axis_name="s"),
    compiler_params=pltpu.CompilerParams(
        needs_layout_passes=False, use_tc_tiling_on_sc=False),
    scratch_types=[
        pltpu.VMEM((ZROWS, H), jnp.float32),
        pltpu.VMEM((KCH,), jnp.int32),
        pltpu.VMEM((NBMAX, BM), jnp.int32),
        pltpu.VMEM((NBMAX, BM), jnp.int32),
        pltpu.VMEM((BM, H), jnp.float32),
        pltpu.VMEM((SUBR,), jnp.int32),
        pltpu.VMEM((2, WR8, H), jnp.float32),
        pltpu.VMEM((2, WR128, 128), jnp.float32),
        pltpu.SemaphoreType.DMA,
        pltpu.SemaphoreType.DMA,
        pltpu.SemaphoreType.DMA,
        pltpu.SemaphoreType.DMA,
        pltpu.SemaphoreType.DMA,
    ],
)(_sc_body)


def kernel(edge_index, edge_attr, num_nodes, W1, b1, W2, b2):
    ebias, keys2d = _mlp_call(
        edge_index, edge_attr, W1, b1.reshape(1, EDGE_DIM),
        W2, b2.reshape(1, H))
    keys = keys2d.reshape(E)
    zeros_in = jnp.zeros((ZROWS, H), jnp.float32)
    _, t_out = _sc_call(keys, ebias, zeros_in)
    out4 = t_out.reshape(N, N // 128, H, 128)
    return out4.transpose(0, 1, 3, 2).reshape(N, N, H)


# speedup vs baseline: 8.0959x; 1.0547x over previous
import functools

import jax
import jax.numpy as jnp
from jax import lax
from jax.experimental import pallas as pl
from jax.experimental.pallas import tpu as pltpu
from jax.experimental.pallas import tpu_sc as plsc

E = 65536
N = 2048
EDGE_DIM = 16
H = 8
KEYS = N * N
NC = 2
NS = 16
NW = NC * NS
RANGE = KEYS // NW
ZROWS = 512
KCH = 8192
MCAP = 8192
BM = 2048
NBMAX = MCAP // BM
SUBR = 16384
WELEMS = 8192
WR8 = WELEMS // 8
WR128 = WELEMS // 128
NWIN = RANGE * H // WELEMS


def _mlp_body(ei_ref, x_ref, w1_ref, b1_ref, w2_ref, b2_ref, eb_ref, key_ref):
    h = jnp.maximum(
        jnp.dot(x_ref[...], w1_ref[...], preferred_element_type=jnp.float32)
        + b1_ref[...], 0.0)
    eb_ref[...] = (
        jnp.dot(h, w2_ref[...], preferred_element_type=jnp.float32)
        + b2_ref[...])
    k = ei_ref[0, :] * N + ei_ref[1, :]
    key_ref[...] = k.reshape(key_ref.shape)


_G = 8
_EB = E // _G


_mlp_call = pl.pallas_call(
    _mlp_body,
    grid=(_G,),
    in_specs=[
        pl.BlockSpec((2, _EB), lambda g: (0, g)),
        pl.BlockSpec((_EB, EDGE_DIM), lambda g: (g, 0)),
        pl.BlockSpec((EDGE_DIM, EDGE_DIM), lambda g: (0, 0)),
        pl.BlockSpec((1, EDGE_DIM), lambda g: (0, 0)),
        pl.BlockSpec((EDGE_DIM, H), lambda g: (0, 0)),
        pl.BlockSpec((1, H), lambda g: (0, 0)),
    ],
    out_specs=[
        pl.BlockSpec((_EB, H), lambda g: (g, 0)),
        pl.BlockSpec((_EB // 128, 128), lambda g: (g, 0)),
    ],
    out_shape=[
        jax.ShapeDtypeStruct((E, H), jnp.float32),
        jax.ShapeDtypeStruct((E // 128, 128), jnp.int32),
    ],
)


_SH16 = 11


def _midx(pos):
    sh = jnp.full((16,), _SH16, jnp.int32)
    mskc = jnp.full((16,), BM - 1, jnp.int32)
    return [pos >> sh, pos & mskc]


def _sc_body(keys_hbm, ebias_hbm, zeros_hbm, scr_hbm, out_hbm,
             zbuf, kbuf, mkeys, mids, rows, tagv, kmask, wbuf, wtbuf,
             zsem, gsem, ssem, rsem, wsem):
    cid = lax.axis_index("c")
    sid = lax.axis_index("s")
    wid = sid * NC + cid
    lo = wid * RANGE
    hi = lo + RANGE

    pltpu.sync_copy(zeros_hbm, zbuf)

    def fire_zero(i, _):
        pltpu.make_async_copy(
            zbuf, scr_hbm.at[pl.ds(lo + i * ZROWS, ZROWS)], zsem).start()
        return 0

    lax.fori_loop(0, RANGE // ZROWS, fire_zero, 0)

    def drain_zero(i, _):
        pltpu.make_async_copy(
            zbuf, scr_hbm.at[pl.ds(lo + i * ZROWS, ZROWS)], zsem).wait()
        return 0

    iota = lax.broadcasted_iota(jnp.int32, (16,), 0)

    lo16 = jnp.full((16,), lo, jnp.int32)
    hi16 = jnp.full((16,), hi, jnp.int32)
    one16 = jnp.full((16,), 1, jnp.int32)
    cap16 = jnp.full((16,), MCAP - 1, jnp.int32)

    def scan_chunk(c, m):
        pltpu.sync_copy(keys_hbm.at[pl.ds(c * KCH, KCH)], kbuf)
        cbase = c * KCH

        @plsc.parallel_loop(0, KCH // 16, unroll=4, carry=m)
        def scan_vreg(j, m):
            k16 = kbuf[pl.ds(j * 16, 16)]
            msk = (k16 >= lo16) & (k16 < hi16)
            c16 = jnp.cumsum(msk.astype(jnp.int32))
            m16 = jnp.full((16,), m, jnp.int32)
            pos = jnp.minimum(m16 + c16 - one16, cap16)
            e16 = jnp.full((16,), cbase + j * 16, jnp.int32) + iota
            plsc.store_scatter(mkeys, _midx(pos), k16, mask=msk)
            plsc.store_scatter(mids, _midx(pos), e16, mask=msk)
            return jnp.minimum(m + jnp.max(c16), MCAP)

        return scan_vreg

    m = lax.fori_loop(0, E // KCH, scan_chunk, jnp.int32(0))

    mm116 = jnp.full((16,), jnp.maximum(m - 1, 0), jnp.int32)
    zero16 = jnp.full((16,), 0, jnp.int32)
    sr16 = jnp.full((16,), SUBR, jnp.int32)
    nv = (m + 15) >> 4

    six16 = jnp.full((16,), 16, jnp.int32)

    def keep_vreg(v, _):
        pos16 = jnp.full((16,), v * 16, jnp.int32) + iota
        valid = pos16 <= mm116
        pos_c = jnp.minimum(pos16, mm116)
        kk = plsc.load_gather(mkeys, _midx(pos_c))
        keep = valid
        for sh in range(1, 16):
            shv = jnp.full((16,), sh, jnp.int32)
            ok = ((iota + shv) < six16) & ((pos16 + shv) <= mm116)
            rot = plsc.load_gather(
                mkeys, _midx(jnp.minimum(pos_c + shv, mm116)))
            keep = keep & ~((rot == kk) & ok)
        kmask[pl.ds(v * 16, 16)] = keep.astype(jnp.int32)
        return 0

    lax.fori_loop(0, nv, keep_vreg, 0)

    for s in range(RANGE // SUBR):
        sublo16 = jnp.full((16,), lo + s * SUBR, jnp.int32)
        subhi16 = sublo16 + sr16

        def tag_store(v, _):
            pos16 = jnp.full((16,), v * 16, jnp.int32) + iota
            valid = pos16 <= mm116
            pos_c = jnp.minimum(pos16, mm116)
            kk = plsc.load_gather(mkeys, _midx(pos_c))
            km16 = kmask[pl.ds(v * 16, 16)]
            insub = (valid & (kk >= sublo16) & (kk < subhi16)
                     & (km16 > zero16))
            kidx = jnp.minimum(jnp.maximum(kk - sublo16, zero16),
                               sr16 - one16)
            plsc.store_scatter(tagv, [kidx], pos16, mask=insub)
            return 0

        lax.fori_loop(0, nv, tag_store, 0)

        @plsc.parallel_loop(0, nv, unroll=4)
        def id_rewrite(v):
            pos16 = jnp.full((16,), v * 16, jnp.int32) + iota
            valid = pos16 <= mm116
            pos_c = jnp.minimum(pos16, mm116)
            kk = plsc.load_gather(mkeys, _midx(pos_c))
            insub = valid & (kk >= sublo16) & (kk < subhi16)
            kidx = jnp.minimum(jnp.maximum(kk - sublo16, zero16),
                               sr16 - one16)
            w16 = plsc.load_gather(tagv, [kidx])
            w_c = jnp.minimum(jnp.maximum(w16, zero16), mm116)
            wid16 = plsc.load_gather(mids, _midx(w_c))
            plsc.store_scatter(mids, _midx(pos_c), wid16, mask=insub)

    key_last = plsc.load_gather(mkeys, _midx(mm116))
    id_last = plsc.load_gather(mids, _midx(mm116))
    nb = (m + BM - 1) >> 11

    def pad_slot(j, _):
        posv = j * 16 + iota
        pmsk = posv >= m
        plsc.store_scatter(mkeys, _midx(posv), key_last, mask=pmsk)
        plsc.store_scatter(mids, _midx(posv), id_last, mask=pmsk)
        return 0

    lax.fori_loop(m >> 4, nb << 7, pad_slot, 0)

    lax.fori_loop(0, RANGE // ZROWS, drain_zero, 0)

    def do_batch(b, _):
        g = pltpu.make_async_copy(ebias_hbm.at[mids.at[b]], rows, gsem)
        g.start()
        g.wait()
        s = pltpu.make_async_copy(rows, scr_hbm.at[mkeys.at[b]], ssem)
        s.start()
        s.wait()
        return 0

    lax.fori_loop(0, nb, do_batch, 0)

    r8_0 = lo
    r128_0 = lo // 16
    patt16 = (iota & jnp.full((16,), 7, jnp.int32)) * jnp.full(
        (16,), 128, jnp.int32) + (iota >> jnp.full((16,), 3, jnp.int32))
    iod8 = iota >> jnp.full((16,), 3, jnp.int32)
    ioc7 = iota & jnp.full((16,), 7, jnp.int32)
    sh7 = jnp.full((16,), 7, jnp.int32)
    m127 = jnp.full((16,), 127, jnp.int32)

    def read_win(w, buf):
        return pltpu.make_async_copy(
            scr_hbm.at[pl.ds(r8_0 + w * WR8, WR8)], buf, rsem)

    def write_win(w, buf):
        return pltpu.make_async_copy(
            buf, out_hbm.at[pl.ds(r128_0 + w * WR128, WR128)], wsem)

    read_win(0, wbuf.at[0]).start()

    def do_window(w, _):
        wb = wbuf.at[w & 1]
        wt = wtbuf.at[w & 1]
        read_win(w, wb).wait()

        @pl.when(w + 1 < NWIN)
        def _():
            read_win(w + 1, wbuf.at[(w + 1) & 1]).start()

        @pl.when(w >= 2)
        def _():
            write_win(w - 2, wtbuf.at[w & 1]).wait()

        @plsc.parallel_loop(0, WELEMS // 16, unroll=8)
        def tx_vreg(v):
            r16 = jnp.full((16,), v * 2, jnp.int32) + iod8
            x16 = plsc.load_gather(wb, [r16, ioc7])
            base = (v >> 6) * 1024 + (v & 63) * 2
            idx16 = patt16 + jnp.full((16,), base, jnp.int32)
            plsc.store_scatter(wt, [idx16 >> sh7, idx16 & m127], x16)

        write_win(w, wt).start()
        return 0

    lax.fori_loop(0, NWIN, do_window, 0)
    write_win(NWIN - 2, wtbuf.at[NWIN & 1]).wait()
    write_win(NWIN - 1, wtbuf.at[(NWIN - 1) & 1]).wait()


_sc_call = functools.partial(
    pl.kernel,
    out_type=[
        jax.ShapeDtypeStruct((KEYS, H), jnp.float32),
        jax.ShapeDtypeStruct((KEYS * H // 128, 128), jnp.float32),
    ],
    mesh=plsc.VectorSubcoreMesh(core_axis_name="c", subcore_axis_name="s"),
    compiler_params=pltpu.CompilerParams(
        needs_layout_passes=False, use_tc_tiling_on_sc=False),
    scratch_types=[
        pltpu.VMEM((ZROWS, H), jnp.float32),
        pltpu.VMEM((KCH,), jnp.int32),
        pltpu.VMEM((NBMAX, BM), jnp.int32),
        pltpu.VMEM((NBMAX, BM), jnp.int32),
        pltpu.VMEM((BM, H), jnp.float32),
        pltpu.VMEM((SUBR,), jnp.int32),
        pltpu.VMEM((MCAP,), jnp.int32),
        pltpu.VMEM((2, WR8, H), jnp.float32),
        pltpu.VMEM((2, WR128, 128), jnp.float32),
        pltpu.SemaphoreType.DMA,
        pltpu.SemaphoreType.DMA,
        pltpu.SemaphoreType.DMA,
        pltpu.SemaphoreType.DMA,
        pltpu.SemaphoreType.DMA,
    ],
)(_sc_body)


def kernel(edge_index, edge_attr, num_nodes, W1, b1, W2, b2):
    ebias, keys2d = _mlp_call(
        edge_index, edge_attr, W1, b1.reshape(1, EDGE_DIM),
        W2, b2.reshape(1, H))
    keys = keys2d.reshape(E)
    zeros_in = jnp.zeros((ZROWS, H), jnp.float32)
    _, t_out = _sc_call(keys, ebias, zeros_in)
    out4 = t_out.reshape(N, N // 128, H, 128)
    return out4.transpose(0, 1, 3, 2).reshape(N, N, H)


# speedup vs baseline: 8.2945x; 1.0245x over previous
import functools

import jax
import jax.numpy as jnp
from jax import lax
from jax.experimental import pallas as pl
from jax.experimental.pallas import tpu as pltpu
from jax.experimental.pallas import tpu_sc as plsc

E = 65536
N = 2048
EDGE_DIM = 16
H = 8
KEYS = N * N
NC = 2
NS = 16
NW = NC * NS
RANGE = KEYS // NW
ZROWS = 512
KCH = 8192
MCAP = 8192
BM = 2048
NBMAX = MCAP // BM
SUBR = 16384
WELEMS = 8192
WR8 = WELEMS // 8
WR128 = WELEMS // 128
NWIN = RANGE * H // WELEMS


def _mlp_body(ei_ref, x_ref, w1_ref, b1_ref, w2_ref, b2_ref, eb_ref, key_ref):
    h = jnp.maximum(
        jnp.dot(x_ref[...], w1_ref[...], preferred_element_type=jnp.float32)
        + b1_ref[...], 0.0)
    eb_ref[...] = (
        jnp.dot(h, w2_ref[...], preferred_element_type=jnp.float32)
        + b2_ref[...])
    k = ei_ref[0, :] * N + ei_ref[1, :]
    key_ref[...] = k.reshape(key_ref.shape)


_G = 8
_EB = E // _G


_mlp_call = pl.pallas_call(
    _mlp_body,
    grid=(_G,),
    in_specs=[
        pl.BlockSpec((2, _EB), lambda g: (0, g)),
        pl.BlockSpec((_EB, EDGE_DIM), lambda g: (g, 0)),
        pl.BlockSpec((EDGE_DIM, EDGE_DIM), lambda g: (0, 0)),
        pl.BlockSpec((1, EDGE_DIM), lambda g: (0, 0)),
        pl.BlockSpec((EDGE_DIM, H), lambda g: (0, 0)),
        pl.BlockSpec((1, H), lambda g: (0, 0)),
    ],
    out_specs=[
        pl.BlockSpec((_EB, H), lambda g: (g, 0)),
        pl.BlockSpec((_EB // 128, 128), lambda g: (g, 0)),
    ],
    out_shape=[
        jax.ShapeDtypeStruct((E, H), jnp.float32),
        jax.ShapeDtypeStruct((E // 128, 128), jnp.int32),
    ],
)


_SH16 = 11


def _midx(pos):
    sh = jnp.full((16,), _SH16, jnp.int32)
    mskc = jnp.full((16,), BM - 1, jnp.int32)
    return [pos >> sh, pos & mskc]


def _sc_body(keys_hbm, ebias_hbm, zeros_hbm, scr_hbm, out_hbm,
             zbuf, kbuf, mkeys, mids, rows, tagv, kmask, wbuf, wtbuf,
             zsem, ksem, gsem, ssem, rsem, wsem):
    cid = lax.axis_index("c")
    sid = lax.axis_index("s")
    wid = sid * NC + cid
    lo = wid * RANGE
    hi = lo + RANGE

    pltpu.sync_copy(zeros_hbm, zbuf)

    def fire_zero(i, _):
        pltpu.make_async_copy(
            zbuf, scr_hbm.at[pl.ds(lo + i * ZROWS, ZROWS)], zsem).start()
        return 0

    lax.fori_loop(0, RANGE // ZROWS, fire_zero, 0)

    def drain_zero(i, _):
        pltpu.make_async_copy(
            zbuf, scr_hbm.at[pl.ds(lo + i * ZROWS, ZROWS)], zsem).wait()
        return 0

    iota = lax.broadcasted_iota(jnp.int32, (16,), 0)

    lo16 = jnp.full((16,), lo, jnp.int32)
    hi16 = jnp.full((16,), hi, jnp.int32)
    one16 = jnp.full((16,), 1, jnp.int32)
    cap16 = jnp.full((16,), MCAP - 1, jnp.int32)

    def read_chunk(c, buf):
        return pltpu.make_async_copy(
            keys_hbm.at[pl.ds(c * KCH, KCH)], buf, ksem)

    read_chunk(0, kbuf.at[0]).start()

    def scan_chunk(c, m):
        kb = kbuf.at[c & 1]
        read_chunk(c, kb).wait()

        @pl.when(c + 1 < E // KCH)
        def _():
            read_chunk(c + 1, kbuf.at[(c + 1) & 1]).start()

        cbase = c * KCH

        @plsc.parallel_loop(0, KCH // 16, unroll=4, carry=m)
        def scan_vreg(j, m):
            k16 = kb[pl.ds(j * 16, 16)]
            msk = (k16 >= lo16) & (k16 < hi16)
            c16 = jnp.cumsum(msk.astype(jnp.int32))
            m16 = jnp.full((16,), m, jnp.int32)
            pos = jnp.minimum(m16 + c16 - one16, cap16)
            e16 = jnp.full((16,), cbase + j * 16, jnp.int32) + iota
            plsc.store_scatter(mkeys, _midx(pos), k16, mask=msk)
            plsc.store_scatter(mids, _midx(pos), e16, mask=msk)
            return jnp.minimum(m + jnp.max(c16), MCAP)

        return scan_vreg

    m = lax.fori_loop(0, E // KCH, scan_chunk, jnp.int32(0))

    mm116 = jnp.full((16,), jnp.maximum(m - 1, 0), jnp.int32)
    zero16 = jnp.full((16,), 0, jnp.int32)
    sr16 = jnp.full((16,), SUBR, jnp.int32)
    nv = (m + 15) >> 4

    six16 = jnp.full((16,), 16, jnp.int32)

    @plsc.parallel_loop(0, nv, unroll=2)
    def keep_vreg(v):
        pos16 = jnp.full((16,), v * 16, jnp.int32) + iota
        valid = pos16 <= mm116
        pos_c = jnp.minimum(pos16, mm116)
        kk = plsc.load_gather(mkeys, _midx(pos_c))
        keep = valid
        for sh in range(1, 16):
            shv = jnp.full((16,), sh, jnp.int32)
            ok = ((iota + shv) < six16) & ((pos16 + shv) <= mm116)
            rot = plsc.load_gather(
                mkeys, _midx(jnp.minimum(pos_c + shv, mm116)))
            keep = keep & ~((rot == kk) & ok)
        kmask[pl.ds(v * 16, 16)] = keep.astype(jnp.int32)

    for s in range(RANGE // SUBR):
        sublo16 = jnp.full((16,), lo + s * SUBR, jnp.int32)
        subhi16 = sublo16 + sr16

        def tag_store(v, _):
            pos16 = jnp.full((16,), v * 16, jnp.int32) + iota
            valid = pos16 <= mm116
            pos_c = jnp.minimum(pos16, mm116)
            kk = plsc.load_gather(mkeys, _midx(pos_c))
            km16 = kmask[pl.ds(v * 16, 16)]
            insub = (valid & (kk >= sublo16) & (kk < subhi16)
                     & (km16 > zero16))
            kidx = jnp.minimum(jnp.maximum(kk - sublo16, zero16),
                               sr16 - one16)
            plsc.store_scatter(tagv, [kidx], pos16, mask=insub)
            return 0

        lax.fori_loop(0, nv, tag_store, 0)

        @plsc.parallel_loop(0, nv, unroll=4)
        def id_rewrite(v):
            pos16 = jnp.full((16,), v * 16, jnp.int32) + iota
            valid = pos16 <= mm116
            pos_c = jnp.minimum(pos16, mm116)
            kk = plsc.load_gather(mkeys, _midx(pos_c))
            insub = valid & (kk >= sublo16) & (kk < subhi16)
            kidx = jnp.minimum(jnp.maximum(kk - sublo16, zero16),
                               sr16 - one16)
            w16 = plsc.load_gather(tagv, [kidx])
            w_c = jnp.minimum(jnp.maximum(w16, zero16), mm116)
            wid16 = plsc.load_gather(mids, _midx(w_c))
            plsc.store_scatter(mids, _midx(pos_c), wid16, mask=insub)

    key_last = plsc.load_gather(mkeys, _midx(mm116))
    id_last = plsc.load_gather(mids, _midx(mm116))
    nb = (m + BM - 1) >> 11

    def pad_slot(j, _):
        posv = j * 16 + iota
        pmsk = posv >= m
        plsc.store_scatter(mkeys, _midx(posv), key_last, mask=pmsk)
        plsc.store_scatter(mids, _midx(posv), id_last, mask=pmsk)
        return 0

    lax.fori_loop(m >> 4, nb << 7, pad_slot, 0)

    lax.fori_loop(0, RANGE // ZROWS, drain_zero, 0)

    def do_batch(b, _):
        g = pltpu.make_async_copy(ebias_hbm.at[mids.at[b]], rows, gsem)
        g.start()
        g.wait()
        s = pltpu.make_async_copy(rows, scr_hbm.at[mkeys.at[b]], ssem)
        s.start()
        s.wait()
        return 0

    lax.fori_loop(0, nb, do_batch, 0)

    r8_0 = lo
    r128_0 = lo // 16
    patt16 = (iota & jnp.full((16,), 7, jnp.int32)) * jnp.full(
        (16,), 128, jnp.int32) + (iota >> jnp.full((16,), 3, jnp.int32))
    iod8 = iota >> jnp.full((16,), 3, jnp.int32)
    ioc7 = iota & jnp.full((16,), 7, jnp.int32)
    sh7 = jnp.full((16,), 7, jnp.int32)
    m127 = jnp.full((16,), 127, jnp.int32)

    def read_win(w, buf):
        return pltpu.make_async_copy(
            scr_hbm.at[pl.ds(r8_0 + w * WR8, WR8)], buf, rsem)

    def write_win(w, buf):
        return pltpu.make_async_copy(
            buf, out_hbm.at[pl.ds(r128_0 + w * WR128, WR128)], wsem)

    read_win(0, wbuf.at[0]).start()

    def do_window(w, _):
        wb = wbuf.at[w & 1]
        wt = wtbuf.at[w & 1]
        read_win(w, wb).wait()

        @pl.when(w + 1 < NWIN)
        def _():
            read_win(w + 1, wbuf.at[(w + 1) & 1]).start()

        @pl.when(w >= 2)
        def _():
            write_win(w - 2, wtbuf.at[w & 1]).wait()

        @plsc.parallel_loop(0, WELEMS // 16, unroll=8)
        def tx_vreg(v):
            r16 = jnp.full((16,), v * 2, jnp.int32) + iod8
            x16 = plsc.load_gather(wb, [r16, ioc7])
            base = (v >> 6) * 1024 + (v & 63) * 2
            idx16 = patt16 + jnp.full((16,), base, jnp.int32)
            plsc.store_scatter(wt, [idx16 >> sh7, idx16 & m127], x16)

        write_win(w, wt).start()
        return 0

    lax.fori_loop(0, NWIN, do_window, 0)
    write_win(NWIN - 2, wtbuf.at[NWIN & 1]).wait()
    write_win(NWIN - 1, wtbuf.at[(NWIN - 1) & 1]).wait()


_sc_call = functools.partial(
    pl.kernel,
    out_type=[
        jax.ShapeDtypeStruct((KEYS, H), jnp.float32),
        jax.ShapeDtypeStruct((KEYS * H // 128, 128), jnp.float32),
    ],
    mesh=plsc.VectorSubcoreMesh(core_axis_name="c", subcore_axis_name="s"),
    compiler_params=pltpu.CompilerParams(
        needs_layout_passes=False, use_tc_tiling_on_sc=False),
    scratch_types=[
        pltpu.VMEM((ZROWS, H), jnp.float32),
        pltpu.VMEM((2, KCH), jnp.int32),
        pltpu.VMEM((NBMAX, BM), jnp.int32),
        pltpu.VMEM((NBMAX, BM), jnp.int32),
        pltpu.VMEM((BM, H), jnp.float32),
        pltpu.VMEM((SUBR,), jnp.int32),
        pltpu.VMEM((MCAP,), jnp.int32),
        pltpu.VMEM((2, WR8, H), jnp.float32),
        pltpu.VMEM((2, WR128, 128), jnp.float32),
        pltpu.SemaphoreType.DMA,
        pltpu.SemaphoreType.DMA,
        pltpu.SemaphoreType.DMA,
        pltpu.SemaphoreType.DMA,
        pltpu.SemaphoreType.DMA,
        pltpu.SemaphoreType.DMA,
    ],
)(_sc_body)


def kernel(edge_index, edge_attr, num_nodes, W1, b1, W2, b2):
    ebias, keys2d = _mlp_call(
        edge_index, edge_attr, W1, b1.reshape(1, EDGE_DIM),
        W2, b2.reshape(1, H))
    keys = keys2d.reshape(E)
    zeros_in = jnp.zeros((ZROWS, H), jnp.float32)
    _, t_out = _sc_call(keys, ebias, zeros_in)
    out4 = t_out.reshape(N, N // 128, H, 128)
    return out4.transpose(0, 1, 3, 2).reshape(N, N, H)
